# baseline (speedup 1.0000x reference)
import functools

import jax
import jax.numpy as jnp
from jax import lax
from jax.experimental import pallas as pl
from jax.experimental.pallas import tpu as pltpu
from jax.experimental.pallas import tpu_sc as plsc

_G = 4
_NBUF = 4
_K = 2


@functools.lru_cache(maxsize=None)
def _make_gather(V, D, batch, ncand):
    info = plsc.get_sparse_core_info()
    NC, NS = info.num_cores, info.num_subcores
    NW = NC * NS
    assert batch % (NW * _G) == 0
    rows = batch // NW
    n = rows // _G
    assert n % _NBUF == 0

    mesh = plsc.VectorSubcoreMesh(core_axis_name="c", subcore_axis_name="s")

    @functools.partial(
        pl.kernel,
        mesh=mesh,
        compiler_params=pltpu.CompilerParams(use_tc_tiling_on_sc=True),
        out_type=jax.ShapeDtypeStruct((batch, ncand, D), jnp.float32),
        scratch_types=(
            [pltpu.VMEM((rows, ncand), jnp.int32)]
            + [pltpu.VMEM((_G, ncand, D), jnp.float32) for _ in range(_NBUF)]
            + [pltpu.SemaphoreType.DMA for _ in range(2 * _NBUF)]
        ),
    )
    def k(idx_hbm, table_hbm, out_hbm, idx_v, *bufs_and_sems):
        bufs = bufs_and_sems[:_NBUF]
        gsem = bufs_and_sems[_NBUF:2 * _NBUF]
        ssem = bufs_and_sems[2 * _NBUF:]
        wid = lax.axis_index("s") * NC + lax.axis_index("c")
        base = wid * rows
        pltpu.sync_copy(idx_hbm.at[pl.ds(base, rows)], idx_v)

        def start_gathers(i, b):
            for g in range(_G):
                pltpu.async_copy(
                    table_hbm.at[idx_v.at[i * _G + g]], bufs[b].at[g], gsem[b])

        def wait_gathers(b):
            pltpu.make_async_copy(
                table_hbm.at[idx_v.at[0]], bufs[b], gsem[b]).wait()

        def start_store(i, b):
            pltpu.async_copy(
                bufs[b], out_hbm.at[pl.ds(base + i * _G, _G)], ssem[b])

        def wait_store(b):
            pltpu.make_async_copy(
                bufs[b], out_hbm.at[pl.ds(base, _G)], ssem[b]).wait()

        for i in range(_K):
            start_gathers(i, i)

        def body(gidx, carry):
            i0 = gidx * _NBUF
            for b in range(_NBUF):
                i = i0 + b
                bg = (b + _K) % _NBUF

                @pl.when(i + _K < n)
                def _():
                    @pl.when(i >= _NBUF - _K)
                    def _():
                        wait_store(bg)
                    start_gathers(i + _K, bg)

                wait_gathers(b)
                start_store(i, b)
            return carry

        lax.fori_loop(0, n // _NBUF, body, 0)

        for j in range(_NBUF - _K):
            wait_store((n - 1 - j) % _NBUF)

    def run(idx, table):
        return k(idx, table)

    return run


def kernel(entity_ids, type_memberships):
    batch, ncand = entity_ids.shape
    V, D = type_memberships.shape
    return _make_gather(V, D, batch, ncand)(
        entity_ids.astype(jnp.int32), type_memberships)


# speedup vs baseline: 1.7940x; 1.7940x over previous
import functools

import jax
import jax.numpy as jnp
from jax import lax
from jax.experimental import pallas as pl
from jax.experimental.pallas import tpu as pltpu
from jax.experimental.pallas import tpu_sc as plsc

_NBUF = 5
_K = 3


@functools.lru_cache(maxsize=None)
def _make_gather(V, D, batch, ncand):
    info = plsc.get_sparse_core_info()
    NC, NS = info.num_cores, info.num_subcores
    NW = NC * NS
    assert batch % NW == 0
    bw = batch // NW
    n = ncand
    assert n % _NBUF == 0 and bw <= 128

    mesh = plsc.VectorSubcoreMesh(core_axis_name="c", subcore_axis_name="s")

    @functools.partial(
        pl.kernel,
        mesh=mesh,
        out_type=jax.ShapeDtypeStruct((ncand, batch, D), jnp.float32),
        scratch_types=(
            [pltpu.VMEM((n, bw), jnp.int32)]
            + [pltpu.VMEM((bw, D), jnp.float32) for _ in range(_NBUF)]
            + [pltpu.SemaphoreType.DMA for _ in range(2 * _NBUF)]
        ),
    )
    def k(idx_hbm, table_hbm, out_hbm, idx_v, *bufs_and_sems):
        bufs = bufs_and_sems[:_NBUF]
        gsem = bufs_and_sems[_NBUF:2 * _NBUF]
        ssem = bufs_and_sems[2 * _NBUF:]
        wid = lax.axis_index("s") * NC + lax.axis_index("c")
        b0 = wid * bw
        pltpu.sync_copy(idx_hbm.at[:, pl.ds(b0, bw)], idx_v)

        def start_gather(i, b):
            pltpu.async_copy(table_hbm.at[idx_v.at[i]], bufs[b], gsem[b])

        def wait_gather(b):
            pltpu.make_async_copy(
                table_hbm.at[idx_v.at[0]], bufs[b], gsem[b]).wait()

        def start_store(i, b):
            pltpu.async_copy(bufs[b], out_hbm.at[i, pl.ds(b0, bw)], ssem[b])

        def wait_store(b):
            pltpu.make_async_copy(
                bufs[b], out_hbm.at[0, pl.ds(b0, bw)], ssem[b]).wait()

        for i in range(_K):
            start_gather(i, i)

        def body(g, carry):
            i0 = g * _NBUF
            for b in range(_NBUF):
                i = i0 + b
                bg = (b + _K) % _NBUF

                @pl.when(i + _K < n)
                def _():
                    @pl.when(i >= _NBUF - _K)
                    def _():
                        wait_store(bg)
                    start_gather(i + _K, bg)

                wait_gather(b)
                start_store(i, b)
            return carry

        lax.fori_loop(0, n // _NBUF, body, 0)

        for j in range(_NBUF - _K):
            wait_store((n - 1 - j) % _NBUF)

    def run(idx, table):
        return k(idx, table)

    return run


def kernel(entity_ids, type_memberships):
    batch, ncand = entity_ids.shape
    V, D = type_memberships.shape
    idx_t = entity_ids.T.astype(jnp.int32)
    out_t = _make_gather(V, D, batch, ncand)(idx_t, type_memberships)
    return out_t.transpose(1, 0, 2)


# speedup vs baseline: 1.7971x; 1.0017x over previous
import functools

import jax
import jax.numpy as jnp
from jax import lax
from jax.experimental import pallas as pl
from jax.experimental.pallas import tpu as pltpu
from jax.experimental.pallas import tpu_sc as plsc

_NBUF = 5
_K = 4


@functools.lru_cache(maxsize=None)
def _make_gather(V, D, batch, ncand):
    info = plsc.get_sparse_core_info()
    NC, NS = info.num_cores, info.num_subcores
    NW = NC * NS
    assert batch % NW == 0
    bw = batch // NW
    n = ncand
    assert n % _NBUF == 0 and bw <= 128

    mesh = plsc.VectorSubcoreMesh(core_axis_name="c", subcore_axis_name="s")

    @functools.partial(
        pl.kernel,
        mesh=mesh,
        out_type=jax.ShapeDtypeStruct((ncand, batch, D), jnp.float32),
        scratch_types=(
            [pltpu.VMEM((n, bw), jnp.int32)]
            + [pltpu.VMEM((bw, D), jnp.float32) for _ in range(_NBUF)]
            + [pltpu.SemaphoreType.DMA for _ in range(2 * _NBUF)]
        ),
    )
    def k(idx_hbm, table_hbm, out_hbm, idx_v, *bufs_and_sems):
        bufs = bufs_and_sems[:_NBUF]
        gsem = bufs_and_sems[_NBUF:2 * _NBUF]
        ssem = bufs_and_sems[2 * _NBUF:]
        wid = lax.axis_index("s") * NC + lax.axis_index("c")
        b0 = wid * bw
        pltpu.sync_copy(idx_hbm.at[:, pl.ds(b0, bw)], idx_v)

        def start_gather(i, b):
            pltpu.async_copy(table_hbm.at[idx_v.at[i]], bufs[b], gsem[b])

        def wait_gather(b):
            pltpu.make_async_copy(
                table_hbm.at[idx_v.at[0]], bufs[b], gsem[b]).wait()

        def start_store(i, b):
            pltpu.async_copy(bufs[b], out_hbm.at[i, pl.ds(b0, bw)], ssem[b])

        def wait_store(b):
            pltpu.make_async_copy(
                bufs[b], out_hbm.at[0, pl.ds(b0, bw)], ssem[b]).wait()

        for i in range(_K):
            start_gather(i, i)

        def body(g, carry):
            i0 = g * _NBUF
            for b in range(_NBUF):
                i = i0 + b
                bg = (b + _K) % _NBUF

                @pl.when(i + _K < n)
                def _():
                    @pl.when(i >= _NBUF - _K)
                    def _():
                        wait_store(bg)
                    start_gather(i + _K, bg)

                wait_gather(b)
                start_store(i, b)
            return carry

        lax.fori_loop(0, n // _NBUF, body, 0)

        for j in range(_NBUF - _K):
            wait_store((n - 1 - j) % _NBUF)

    def run(idx, table):
        return k(idx, table)

    return run


def kernel(entity_ids, type_memberships):
    batch, ncand = entity_ids.shape
    V, D = type_memberships.shape
    idx_t = entity_ids.T.astype(jnp.int32)
    out_t = _make_gather(V, D, batch, ncand)(idx_t, type_memberships)
    return out_t.transpose(1, 0, 2)
